# baseline (speedup 1.0000x reference)
import functools

import jax
import jax.numpy as jnp
from jax import lax
from jax.experimental import pallas as pl
from jax.experimental.pallas import tpu as pltpu
from jax.experimental.pallas import tpu_sc as plsc

N_NODES = 10000
N_EDGES = 320000
NCHUNK = 2
EC = N_EDGES // NCHUNK
LATENT = 128
BE = 3200
NEBC = EC // BE
NSLOPE = 0.01


def _leaky(x):
    return jnp.where(x >= 0, x, NSLOPE * x)


HALF = LATENT // 2


def _pack_pairs(x):
    b = jax.lax.bitcast_convert_type(x.astype(jnp.bfloat16), jnp.uint16)
    lo = b[:, :HALF].astype(jnp.uint32)
    hi = b[:, HALF:].astype(jnp.uint32)
    return jax.lax.bitcast_convert_type(lo | (hi << 16), jnp.float32)


def _unpack_sum(w0, w1):
    u0 = jax.lax.bitcast_convert_type(w0, jnp.uint32)
    u1 = jax.lax.bitcast_convert_type(w1, jnp.uint32)
    e = (jax.lax.bitcast_convert_type(u0 << 16, jnp.float32)
         + jax.lax.bitcast_convert_type(u1 << 16, jnp.float32))
    o = (jax.lax.bitcast_convert_type(u0 & jnp.uint32(0xFFFF0000), jnp.float32)
         + jax.lax.bitcast_convert_type(u1 & jnp.uint32(0xFFFF0000), jnp.float32))
    return jnp.concatenate([e, o], axis=-1)


def _deint_cols(W):
    return jnp.concatenate([W[:, 0::2], W[:, 1::2]], axis=1)


def _deint_rows(W):
    return jnp.concatenate([W[0::2], W[1::2]], axis=0)


def _embed_nodes_body(n_ref, nw_ref, nb_ref, g_ref, gw_ref, gb_ref,
                      no_ref, go_ref):
    no_ref[...] = jnp.dot(n_ref[...], nw_ref[...],
                          preferred_element_type=jnp.float32) + nb_ref[...]
    go_ref[...] = jnp.dot(g_ref[...], gw_ref[...],
                          preferred_element_type=jnp.float32) + gb_ref[...]


def _embed_nodes(nodes, nW, nb, g, gW, gb):
    return pl.pallas_call(
        _embed_nodes_body,
        out_shape=[
            jax.ShapeDtypeStruct((N_NODES, LATENT), jnp.float32),
            jax.ShapeDtypeStruct((1, LATENT), jnp.float32),
        ],
    )(nodes, nW, nb.reshape(1, LATENT), g, gW, gb.reshape(1, LATENT))


def _prep_body(n_ref, g_ref, w2_ref, wg_ref, b0_ref,
               tab_ref, ce_ref):
    n = n_ref[...]
    both = jnp.dot(n, w2_ref[...], precision=jax.lax.Precision.HIGHEST,
                   preferred_element_type=jnp.float32)
    tab_ref[:N_NODES, :] = both[:, :LATENT]
    tab_ref[N_NODES:, :] = both[:, LATENT:]
    ce_ref[...] = jnp.dot(g_ref[...], wg_ref[...],
                          preferred_element_type=jnp.float32) + b0_ref[...]


def _prep(nodes, g, Ws, Wr, Wg, b0):
    W2 = jnp.concatenate([Ws, Wr], axis=1)
    return pl.pallas_call(
        _prep_body,
        out_shape=[
            jax.ShapeDtypeStruct((2 * N_NODES, LATENT), jnp.float32),
            jax.ShapeDtypeStruct((1, LATENT), jnp.float32),
        ],
    )(nodes, g, W2, Wg, b0.reshape(1, LATENT))


def _prep_first_body(n_ref, g_ref, w2_ref, wg_ref, b0_ref,
                     embw_ref, embb_ref, we_ref,
                     tab_ref, ce_ref, weff_ref):
    n = n_ref[...]
    both = jnp.dot(n, w2_ref[...], precision=jax.lax.Precision.HIGHEST,
                   preferred_element_type=jnp.float32)
    tab_ref[:N_NODES, :] = both[:, :LATENT]
    tab_ref[N_NODES:, :] = both[:, LATENT:]
    we = we_ref[...]
    hi = jax.lax.Precision.HIGHEST
    ce_ref[...] = (jnp.dot(g_ref[...], wg_ref[...],
                           preferred_element_type=jnp.float32) + b0_ref[...]
                   + jnp.dot(embb_ref[...], we, precision=hi,
                             preferred_element_type=jnp.float32))
    weff_ref[...] = jnp.dot(embw_ref[...], we, precision=hi,
                            preferred_element_type=jnp.float32)


def _prep_first(nodes, g, Ws, Wr, Wg, b0, embW, embb, We):
    W2 = jnp.concatenate([Ws, Wr], axis=1)
    return pl.pallas_call(
        _prep_first_body,
        out_shape=[
            jax.ShapeDtypeStruct((2 * N_NODES, LATENT), jnp.float32),
            jax.ShapeDtypeStruct((1, LATENT), jnp.float32),
            jax.ShapeDtypeStruct((16, LATENT), jnp.float32),
        ],
    )(nodes, g, W2, Wg, b0.reshape(1, LATENT),
      embW, embb.reshape(1, LATENT), We)


@functools.cache
def _sc_mesh():
    return plsc.VectorSubcoreMesh(core_axis_name="c", subcore_axis_name="s")


GW = 256
SW = 128
NSUB = 16
NPS = 632
N_NODES_PAD = NPS * NSUB


def _sc_gather_body(tab_hbm, idx_hbm, out_hbm):
    def body(i_vmem, o_vmem):
        pltpu.sync_copy(tab_hbm.at[i_vmem.at[0]], o_vmem)

    pltpu.emit_pipeline(
        body,
        grid=(2 * EC // GW,),
        in_specs=[pl.BlockSpec((1, GW), lambda i: (0, i))],
        out_specs=[pl.BlockSpec((GW, LATENT), lambda i: (i, 0))],
        core_axis_name=("c", "s"),
        dimension_semantics=(pltpu.PARALLEL,),
    )(idx_hbm, out_hbm)


def _gather2(tab, idx2d):
    f = pl.kernel(
        _sc_gather_body,
        out_type=jax.ShapeDtypeStruct((2 * EC, LATENT), jnp.float32),
        mesh=_sc_mesh(),
    )
    return f(tab, idx2d)


def _sc_segsum_body(edges_hbm, snd_hbm, rcv_hbm, zeros_hbm,
                    sent_hbm, recv_hbm, acc):
    c = lax.axis_index("c")
    s = lax.axis_index("s")

    pltpu.sync_copy(zeros_hbm, acc.at[pl.ds(s * NPS, NPS)])
    plsc.subcore_barrier()

    def body(e_vmem, i_vmem):
        pltpu.sync_copy(e_vmem, acc.at[i_vmem.at[0]], add=True)

    pipe = pltpu.emit_pipeline(
        body,
        grid=(EC // SW,),
        in_specs=[
            pl.BlockSpec((SW, LATENT), lambda i: (i, 0)),
            pl.BlockSpec((1, SW), lambda i: (0, i)),
        ],
        out_specs=[],
        core_axis_name="s",
        dimension_semantics=(pltpu.PARALLEL,),
    )

    @pl.when(c == 0)
    def _():
        pipe(edges_hbm, snd_hbm)

    @pl.when(c == 1)
    def _():
        pipe(edges_hbm, rcv_hbm)

    plsc.subcore_barrier()
    last = N_NODES - 15 * NPS

    @pl.when(c == 0)
    def _():
        @pl.when(s < 15)
        def _():
            rows = pl.ds(s * NPS, NPS)
            pltpu.sync_copy(acc.at[rows], sent_hbm.at[rows])

        @pl.when(s == 15)
        def _():
            rows = pl.ds(15 * NPS, last)
            pltpu.sync_copy(acc.at[rows], sent_hbm.at[rows])

    @pl.when(c == 1)
    def _():
        @pl.when(s < 15)
        def _():
            rows = pl.ds(s * NPS, NPS)
            pltpu.sync_copy(acc.at[rows], recv_hbm.at[rows])

        @pl.when(s == 15)
        def _():
            rows = pl.ds(15 * NPS, last)
            pltpu.sync_copy(acc.at[rows], recv_hbm.at[rows])


def _segsum2(edges_chunk, snd2d, rcv2d):
    zeros = jnp.zeros((NPS, LATENT), jnp.float32)
    f = pl.kernel(
        _sc_segsum_body,
        out_type=[
            jax.ShapeDtypeStruct((N_NODES, LATENT), jnp.float32),
            jax.ShapeDtypeStruct((N_NODES, LATENT), jnp.float32),
        ],
        mesh=_sc_mesh(),
        scratch_types=[
            pltpu.VMEM_SHARED((N_NODES_PAD, LATENT), jnp.float32),
        ],
    )
    return f(edges_chunk, snd2d, rcv2d, zeros)


def _edge_body(e_ref, g_ref, we_ref, w1_ref, ce_ref, b1_ref,
               out_ref, agg_ref, transposed_e):
    if transposed_e:
        e = e_ref[...].T
    else:
        e = e_ref[...]
    h = jnp.dot(e, we_ref[...], precision=jax.lax.Precision.HIGHEST,
                preferred_element_type=jnp.float32)
    h = h + g_ref[0] + g_ref[1] + ce_ref[...]
    h = _leaky(h)
    o = jnp.dot(h, w1_ref[...], precision=jax.lax.Precision.HIGHEST,
                preferred_element_type=jnp.float32) + b1_ref[...]
    out_ref[...] = o

    @pl.when(pl.program_id(0) == 0)
    def _():
        agg_ref[...] = jnp.zeros_like(agg_ref)

    agg_ref[...] += jnp.sum(o, axis=0, keepdims=True)


def _edge_update(edges, G, We, W1, ce, b1, transposed_e=False, col0=0):
    cb = col0 // BE
    if transposed_e:
        espec = pl.BlockSpec((edges.shape[0], BE), lambda i: (0, i + cb))
    else:
        espec = pl.BlockSpec((BE, LATENT), lambda i: (i, 0))
    return pl.pallas_call(
        functools.partial(_edge_body, transposed_e=transposed_e),
        grid=(NEBC,),
        in_specs=[
            espec,
            pl.BlockSpec((2, BE, LATENT), lambda i: (0, i, 0)),
            pl.BlockSpec((edges.shape[0] if transposed_e else LATENT, LATENT),
                         lambda i: (0, 0)),
            pl.BlockSpec((LATENT, LATENT), lambda i: (0, 0)),
            pl.BlockSpec((1, LATENT), lambda i: (0, 0)),
            pl.BlockSpec((1, LATENT), lambda i: (0, 0)),
        ],
        out_specs=[
            pl.BlockSpec((BE, LATENT), lambda i: (i, 0)),
            pl.BlockSpec((1, LATENT), lambda i: (0, 0)),
        ],
        out_shape=[
            jax.ShapeDtypeStruct((EC, LATENT), jnp.float32),
            jax.ShapeDtypeStruct((1, LATENT), jnp.float32),
        ],
    )(edges, G.reshape(2, EC, LATENT), We, W1, ce,
      b1.reshape(1, LATENT))


def _node_body(n_ref, sa_ref, sb_ref, ra_ref, rb_ref,
               wn_ref, ws_ref, wr_ref, cn_ref, w1_ref, b1_ref,
               eaa_ref, eab_ref, g_ref, wa_ref, wb_ref, wc_ref, bg_ref,
               wg1_ref, bg1_ref,
               out_ref, na_ref, go_ref, nblocks):
    h = (jnp.dot(n_ref[...], wn_ref[...], preferred_element_type=jnp.float32)
         + jnp.dot(sa_ref[...] + sb_ref[...], ws_ref[...],
                   preferred_element_type=jnp.float32)
         + jnp.dot(ra_ref[...] + rb_ref[...], wr_ref[...],
                   preferred_element_type=jnp.float32)
         + cn_ref[...])
    h = _leaky(h)
    o = jnp.dot(h, w1_ref[...], preferred_element_type=jnp.float32) + b1_ref[...]
    out_ref[...] = o

    @pl.when(pl.program_id(0) == 0)
    def _():
        na_ref[...] = jnp.zeros_like(na_ref)

    na_ref[...] += jnp.sum(o, axis=0, keepdims=True)

    @pl.when(pl.program_id(0) == nblocks - 1)
    def _():
        ea = eaa_ref[...] + eab_ref[...]
        hg = (jnp.dot(na_ref[...], wa_ref[...],
                      preferred_element_type=jnp.float32)
              + jnp.dot(ea, wb_ref[...], preferred_element_type=jnp.float32)
              + jnp.dot(g_ref[...], wc_ref[...],
                        preferred_element_type=jnp.float32)
              + bg_ref[...])
        hg = _leaky(hg)
        go_ref[...] = jnp.dot(hg, wg1_ref[...],
                              preferred_element_type=jnp.float32) + bg1_ref[...]


def _node_update(nodes, sA, sB, rA, rB, Wn, Wse, Wre, cn, W1, b1,
                 eaA, eaB, g, Wa, Wb, Wc, bg0, Wg1, bg1):
    BN = 2000
    nb = N_NODES // BN
    blk = lambda i: (i, 0)
    full = lambda i: (0, 0)
    return pl.pallas_call(
        functools.partial(_node_body, nblocks=nb),
        grid=(nb,),
        in_specs=(
            [pl.BlockSpec((BN, LATENT), blk) for _ in range(5)]
            + [pl.BlockSpec((LATENT, LATENT), full) for _ in range(3)]
            + [pl.BlockSpec((1, LATENT), full),
               pl.BlockSpec((LATENT, LATENT), full),
               pl.BlockSpec((1, LATENT), full),
               pl.BlockSpec((1, LATENT), full),
               pl.BlockSpec((1, LATENT), full),
               pl.BlockSpec((1, LATENT), full)]
            + [pl.BlockSpec((LATENT, LATENT), full) for _ in range(3)]
            + [pl.BlockSpec((1, LATENT), full),
               pl.BlockSpec((LATENT, LATENT), full),
               pl.BlockSpec((1, LATENT), full)]
        ),
        out_specs=[
            pl.BlockSpec((BN, LATENT), blk),
            pl.BlockSpec((1, LATENT), full),
            pl.BlockSpec((1, LATENT), full),
        ],
        out_shape=[
            jax.ShapeDtypeStruct((N_NODES, LATENT), jnp.float32),
            jax.ShapeDtypeStruct((1, LATENT), jnp.float32),
            jax.ShapeDtypeStruct((1, LATENT), jnp.float32),
        ],
    )(nodes, sA, sB, rA, rB, Wn, Wse, Wre, cn, W1, b1.reshape(1, LATENT),
      eaA, eaB, g, Wa, Wb, Wc, bg0.reshape(1, LATENT), Wg1,
      bg1.reshape(1, LATENT))


def kernel(nodes, edges, globals_, senders, receivers,
           emb_node_W, emb_node_b, emb_edge_W, emb_edge_b,
           emb_global_W, emb_global_b,
           edge0_W0, edge0_b0, edge0_W1, edge0_b1,
           node0_W0, node0_b0, node0_W1, node0_b1,
           glob0_W0, glob0_b0, glob0_W1, glob0_b1,
           edge1_W0, edge1_b0, edge1_W1, edge1_b1,
           node1_W0, node1_b0, node1_W1, node1_b1,
           glob1_W0, glob1_b0, glob1_W1, glob1_b1):
    p = locals()
    rshift = receivers + N_NODES
    gidx = [jnp.concatenate([senders[c * EC:(c + 1) * EC],
                             rshift[c * EC:(c + 1) * EC]]).reshape(1, 2 * EC)
            for c in range(NCHUNK)]
    sidx = [senders[c * EC:(c + 1) * EC].reshape(1, EC)
            for c in range(NCHUNK)]
    ridx = [receivers[c * EC:(c + 1) * EC].reshape(1, EC)
            for c in range(NCHUNK)]
    edgesT = edges.T
    nodes_l, g_l = _embed_nodes(nodes, emb_node_W, emb_node_b,
                                globals_, emb_global_W, emb_global_b)
    e_chunks = [None, None]
    for s in range(2):
        W0 = p[f'edge{s}_W0']
        We, Ws, Wr, Wg = (W0[i * LATENT:(i + 1) * LATENT] for i in range(4))
        We_p = We
        Wg_p = Wg
        b0_p = p[f'edge{s}_b0']
        W1_p = p[f'edge{s}_W1']
        if s == 0:
            tab, ce, Weff = _prep_first(nodes_l, g_l, Ws, Wr, Wg_p,
                                        b0_p, emb_edge_W,
                                        emb_edge_b, We_p)
        else:
            tab, ce = _prep(nodes_l, g_l, Ws, Wr, Wg_p, b0_p)
        G = [_gather2(tab, gidx[c]) for c in range(NCHUNK)]
        ea = [None, None]
        for c in range(NCHUNK):
            if s == 0:
                e_chunks[c], ea[c] = _edge_update(
                    edgesT, G[c], Weff, W1_p, ce, p['edge0_b1'],
                    transposed_e=True, col0=c * EC)
            else:
                e_chunks[c], ea[c] = _edge_update(
                    e_chunks[c], G[c], We_p, W1_p, ce,
                    p[f'edge{s}_b1'])
        ss = [_segsum2(e_chunks[c], sidx[c], ridx[c]) for c in range(NCHUNK)]
        W0n = p[f'node{s}_W0']
        Wn, Wse, Wre, Wgn = (W0n[i * LATENT:(i + 1) * LATENT] for i in range(4))
        cn = jnp.dot(g_l, Wgn) + p[f'node{s}_b0'].reshape(1, LATENT)
        W0g = p[f'glob{s}_W0']
        Wa, Wb, Wc = (W0g[i * LATENT:(i + 1) * LATENT] for i in range(3))
        nodes_l, na, g_l = _node_update(
            nodes_l, ss[0][0], ss[1][0], ss[0][1], ss[1][1],
            Wn, Wse, Wre, cn, p[f'node{s}_W1'], p[f'node{s}_b1'],
            ea[0], ea[1], g_l, Wa, Wb, Wc, p[f'glob{s}_b0'],
            p[f'glob{s}_W1'], p[f'glob{s}_b1'])
    edges_out = jnp.concatenate(e_chunks, axis=0)
    return nodes_l, edges_out, g_l


# speedup vs baseline: 1.3316x; 1.3316x over previous
import functools

import jax
import jax.numpy as jnp
from jax import lax
from jax.experimental import pallas as pl
from jax.experimental.pallas import tpu as pltpu
from jax.experimental.pallas import tpu_sc as plsc

N_NODES = 10000
N_EDGES = 320000
NCHUNK = 2
EC = N_EDGES // NCHUNK
LATENT = 128
BE = 3200
NEBC = EC // BE
NSLOPE = 0.01


def _leaky(x):
    return jnp.where(x >= 0, x, NSLOPE * x)


def _embed_nodes_body(n_ref, nw_ref, nb_ref, g_ref, gw_ref, gb_ref,
                      no_ref, go_ref):
    no_ref[...] = jnp.dot(n_ref[...], nw_ref[...],
                          preferred_element_type=jnp.float32) + nb_ref[...]
    go_ref[...] = jnp.dot(g_ref[...], gw_ref[...],
                          preferred_element_type=jnp.float32) + gb_ref[...]


def _embed_nodes(nodes, nW, nb, g, gW, gb):
    return pl.pallas_call(
        _embed_nodes_body,
        out_shape=[
            jax.ShapeDtypeStruct((N_NODES, LATENT), jnp.float32),
            jax.ShapeDtypeStruct((1, LATENT), jnp.float32),
        ],
    )(nodes, nW, nb.reshape(1, LATENT), g, gW, gb.reshape(1, LATENT))


def _prep_body(n_ref, g_ref, w2_ref, wg_ref, b0_ref,
               tab_ref, ce_ref):
    n = n_ref[...]
    both = jnp.dot(n, w2_ref[...], precision=jax.lax.Precision.HIGHEST,
                   preferred_element_type=jnp.float32)
    tab_ref[:N_NODES, :] = both[:, :LATENT]
    tab_ref[N_NODES:, :] = both[:, LATENT:]
    ce_ref[...] = jnp.dot(g_ref[...], wg_ref[...],
                          preferred_element_type=jnp.float32) + b0_ref[...]


def _prep(nodes, g, Ws, Wr, Wg, b0):
    W2 = jnp.concatenate([Ws, Wr], axis=1)
    return pl.pallas_call(
        _prep_body,
        out_shape=[
            jax.ShapeDtypeStruct((2 * N_NODES, LATENT), jnp.float32),
            jax.ShapeDtypeStruct((1, LATENT), jnp.float32),
        ],
    )(nodes, g, W2, Wg, b0.reshape(1, LATENT))


def _prep_first_body(n_ref, g_ref, w2_ref, wg_ref, b0_ref,
                     embw_ref, embb_ref, we_ref,
                     tab_ref, ce_ref, weff_ref):
    n = n_ref[...]
    both = jnp.dot(n, w2_ref[...], precision=jax.lax.Precision.HIGHEST,
                   preferred_element_type=jnp.float32)
    tab_ref[:N_NODES, :] = both[:, :LATENT]
    tab_ref[N_NODES:, :] = both[:, LATENT:]
    we = we_ref[...]
    hi = jax.lax.Precision.HIGHEST
    ce_ref[...] = (jnp.dot(g_ref[...], wg_ref[...],
                           preferred_element_type=jnp.float32) + b0_ref[...]
                   + jnp.dot(embb_ref[...], we, precision=hi,
                             preferred_element_type=jnp.float32))
    weff_ref[...] = jnp.dot(embw_ref[...], we, precision=hi,
                            preferred_element_type=jnp.float32)


def _prep_first(nodes, g, Ws, Wr, Wg, b0, embW, embb, We):
    W2 = jnp.concatenate([Ws, Wr], axis=1)
    return pl.pallas_call(
        _prep_first_body,
        out_shape=[
            jax.ShapeDtypeStruct((2 * N_NODES, LATENT), jnp.float32),
            jax.ShapeDtypeStruct((1, LATENT), jnp.float32),
            jax.ShapeDtypeStruct((16, LATENT), jnp.float32),
        ],
    )(nodes, g, W2, Wg, b0.reshape(1, LATENT),
      embW, embb.reshape(1, LATENT), We)


@functools.cache
def _sc_mesh():
    return plsc.VectorSubcoreMesh(core_axis_name="c", subcore_axis_name="s")


GW = 256
SW = 128
NSUB = 16
NPS = 632
N_NODES_PAD = NPS * NSUB


def _sc_gather_body(tab_hbm, idx_hbm, out_hbm):
    def body(i_vmem, o_vmem):
        pltpu.sync_copy(tab_hbm.at[i_vmem.at[0]], o_vmem)

    pltpu.emit_pipeline(
        body,
        grid=(2 * EC // GW,),
        in_specs=[pl.BlockSpec((1, GW), lambda i: (0, i))],
        out_specs=[pl.BlockSpec((GW, LATENT), lambda i: (i, 0))],
        core_axis_name=("c", "s"),
        dimension_semantics=(pltpu.PARALLEL,),
    )(idx_hbm, out_hbm)


def _gather2(tab, idx2d):
    f = pl.kernel(
        _sc_gather_body,
        out_type=jax.ShapeDtypeStruct((2 * EC, LATENT), jnp.float32),
        mesh=_sc_mesh(),
    )
    return f(tab, idx2d)


def _sc_segsum_body(edges_hbm, snd_hbm, rcv_hbm, zeros_hbm,
                    sent_hbm, recv_hbm, acc):
    c = lax.axis_index("c")
    s = lax.axis_index("s")

    pltpu.sync_copy(zeros_hbm, acc.at[pl.ds(s * NPS, NPS)])
    plsc.subcore_barrier()

    def body(e_vmem, i_vmem):
        pltpu.sync_copy(e_vmem, acc.at[i_vmem.at[0]], add=True)

    pipe = pltpu.emit_pipeline(
        body,
        grid=(EC // SW,),
        in_specs=[
            pl.BlockSpec((SW, LATENT), lambda i: (i, 0)),
            pl.BlockSpec((1, SW), lambda i: (0, i)),
        ],
        out_specs=[],
        core_axis_name="s",
        dimension_semantics=(pltpu.PARALLEL,),
    )

    @pl.when(c == 0)
    def _():
        pipe(edges_hbm, snd_hbm)

    @pl.when(c == 1)
    def _():
        pipe(edges_hbm, rcv_hbm)

    plsc.subcore_barrier()
    last = N_NODES - 15 * NPS

    @pl.when(c == 0)
    def _():
        @pl.when(s < 15)
        def _():
            rows = pl.ds(s * NPS, NPS)
            pltpu.sync_copy(acc.at[rows], sent_hbm.at[rows])

        @pl.when(s == 15)
        def _():
            rows = pl.ds(15 * NPS, last)
            pltpu.sync_copy(acc.at[rows], sent_hbm.at[rows])

    @pl.when(c == 1)
    def _():
        @pl.when(s < 15)
        def _():
            rows = pl.ds(s * NPS, NPS)
            pltpu.sync_copy(acc.at[rows], recv_hbm.at[rows])

        @pl.when(s == 15)
        def _():
            rows = pl.ds(15 * NPS, last)
            pltpu.sync_copy(acc.at[rows], recv_hbm.at[rows])


def _segsum2(edges_chunk, snd2d, rcv2d):
    zeros = jnp.zeros((NPS, LATENT), jnp.float32)
    f = pl.kernel(
        _sc_segsum_body,
        out_type=[
            jax.ShapeDtypeStruct((N_NODES, LATENT), jnp.float32),
            jax.ShapeDtypeStruct((N_NODES, LATENT), jnp.float32),
        ],
        mesh=_sc_mesh(),
        scratch_types=[
            pltpu.VMEM_SHARED((N_NODES_PAD, LATENT), jnp.float32),
        ],
    )
    return f(edges_chunk, snd2d, rcv2d, zeros)


def _edge_body(e_ref, g_ref, we_ref, w1_ref, ce_ref, b1_ref,
               out_ref, agg_ref, transposed_e):
    if transposed_e:
        e = e_ref[...].T
    else:
        e = e_ref[...]
    h = jnp.dot(e, we_ref[...], preferred_element_type=jnp.float32)
    h = h + g_ref[0] + g_ref[1] + ce_ref[...]
    h = _leaky(h)
    o = jnp.dot(h, w1_ref[...], preferred_element_type=jnp.float32) + b1_ref[...]
    out_ref[...] = o

    @pl.when(pl.program_id(0) == 0)
    def _():
        agg_ref[...] = jnp.zeros_like(agg_ref)

    agg_ref[...] += jnp.sum(o, axis=0, keepdims=True)


def _edge_update(edges, G, We, W1, ce, b1, transposed_e=False, col0=0):
    cb = col0 // BE
    if transposed_e:
        espec = pl.BlockSpec((edges.shape[0], BE), lambda i: (0, i + cb))
    else:
        espec = pl.BlockSpec((BE, LATENT), lambda i: (i, 0))
    return pl.pallas_call(
        functools.partial(_edge_body, transposed_e=transposed_e),
        grid=(NEBC,),
        in_specs=[
            espec,
            pl.BlockSpec((2, BE, LATENT), lambda i: (0, i, 0)),
            pl.BlockSpec((edges.shape[0] if transposed_e else LATENT, LATENT),
                         lambda i: (0, 0)),
            pl.BlockSpec((LATENT, LATENT), lambda i: (0, 0)),
            pl.BlockSpec((1, LATENT), lambda i: (0, 0)),
            pl.BlockSpec((1, LATENT), lambda i: (0, 0)),
        ],
        out_specs=[
            pl.BlockSpec((BE, LATENT), lambda i: (i, 0)),
            pl.BlockSpec((1, LATENT), lambda i: (0, 0)),
        ],
        out_shape=[
            jax.ShapeDtypeStruct((EC, LATENT), jnp.float32),
            jax.ShapeDtypeStruct((1, LATENT), jnp.float32),
        ],
    )(edges, G.reshape(2, EC, LATENT), We, W1, ce,
      b1.reshape(1, LATENT))


def _node_body(n_ref, sa_ref, sb_ref, ra_ref, rb_ref,
               wn_ref, ws_ref, wr_ref, cn_ref, w1_ref, b1_ref,
               eaa_ref, eab_ref, g_ref, wa_ref, wb_ref, wc_ref, bg_ref,
               wg1_ref, bg1_ref,
               out_ref, na_ref, go_ref, nblocks):
    h = (jnp.dot(n_ref[...], wn_ref[...], preferred_element_type=jnp.float32)
         + jnp.dot(sa_ref[...] + sb_ref[...], ws_ref[...],
                   preferred_element_type=jnp.float32)
         + jnp.dot(ra_ref[...] + rb_ref[...], wr_ref[...],
                   preferred_element_type=jnp.float32)
         + cn_ref[...])
    h = _leaky(h)
    o = jnp.dot(h, w1_ref[...], preferred_element_type=jnp.float32) + b1_ref[...]
    out_ref[...] = o

    @pl.when(pl.program_id(0) == 0)
    def _():
        na_ref[...] = jnp.zeros_like(na_ref)

    na_ref[...] += jnp.sum(o, axis=0, keepdims=True)

    @pl.when(pl.program_id(0) == nblocks - 1)
    def _():
        ea = eaa_ref[...] + eab_ref[...]
        hg = (jnp.dot(na_ref[...], wa_ref[...],
                      preferred_element_type=jnp.float32)
              + jnp.dot(ea, wb_ref[...], preferred_element_type=jnp.float32)
              + jnp.dot(g_ref[...], wc_ref[...],
                        preferred_element_type=jnp.float32)
              + bg_ref[...])
        hg = _leaky(hg)
        go_ref[...] = jnp.dot(hg, wg1_ref[...],
                              preferred_element_type=jnp.float32) + bg1_ref[...]


def _node_update(nodes, sA, sB, rA, rB, Wn, Wse, Wre, cn, W1, b1,
                 eaA, eaB, g, Wa, Wb, Wc, bg0, Wg1, bg1):
    BN = 2000
    nb = N_NODES // BN
    blk = lambda i: (i, 0)
    full = lambda i: (0, 0)
    return pl.pallas_call(
        functools.partial(_node_body, nblocks=nb),
        grid=(nb,),
        in_specs=(
            [pl.BlockSpec((BN, LATENT), blk) for _ in range(5)]
            + [pl.BlockSpec((LATENT, LATENT), full) for _ in range(3)]
            + [pl.BlockSpec((1, LATENT), full),
               pl.BlockSpec((LATENT, LATENT), full),
               pl.BlockSpec((1, LATENT), full),
               pl.BlockSpec((1, LATENT), full),
               pl.BlockSpec((1, LATENT), full),
               pl.BlockSpec((1, LATENT), full)]
            + [pl.BlockSpec((LATENT, LATENT), full) for _ in range(3)]
            + [pl.BlockSpec((1, LATENT), full),
               pl.BlockSpec((LATENT, LATENT), full),
               pl.BlockSpec((1, LATENT), full)]
        ),
        out_specs=[
            pl.BlockSpec((BN, LATENT), blk),
            pl.BlockSpec((1, LATENT), full),
            pl.BlockSpec((1, LATENT), full),
        ],
        out_shape=[
            jax.ShapeDtypeStruct((N_NODES, LATENT), jnp.float32),
            jax.ShapeDtypeStruct((1, LATENT), jnp.float32),
            jax.ShapeDtypeStruct((1, LATENT), jnp.float32),
        ],
    )(nodes, sA, sB, rA, rB, Wn, Wse, Wre, cn, W1, b1.reshape(1, LATENT),
      eaA, eaB, g, Wa, Wb, Wc, bg0.reshape(1, LATENT), Wg1,
      bg1.reshape(1, LATENT))


def kernel(nodes, edges, globals_, senders, receivers,
           emb_node_W, emb_node_b, emb_edge_W, emb_edge_b,
           emb_global_W, emb_global_b,
           edge0_W0, edge0_b0, edge0_W1, edge0_b1,
           node0_W0, node0_b0, node0_W1, node0_b1,
           glob0_W0, glob0_b0, glob0_W1, glob0_b1,
           edge1_W0, edge1_b0, edge1_W1, edge1_b1,
           node1_W0, node1_b0, node1_W1, node1_b1,
           glob1_W0, glob1_b0, glob1_W1, glob1_b1):
    p = locals()
    rshift = receivers + N_NODES
    gidx = [jnp.concatenate([senders[c * EC:(c + 1) * EC],
                             rshift[c * EC:(c + 1) * EC]]).reshape(1, 2 * EC)
            for c in range(NCHUNK)]
    sidx = [senders[c * EC:(c + 1) * EC].reshape(1, EC)
            for c in range(NCHUNK)]
    ridx = [receivers[c * EC:(c + 1) * EC].reshape(1, EC)
            for c in range(NCHUNK)]
    edgesT = edges.T
    nodes_l, g_l = _embed_nodes(nodes, emb_node_W, emb_node_b,
                                globals_, emb_global_W, emb_global_b)
    e_chunks = [None, None]
    for s in range(2):
        W0 = p[f'edge{s}_W0']
        We, Ws, Wr, Wg = (W0[i * LATENT:(i + 1) * LATENT] for i in range(4))
        We_p = We
        Wg_p = Wg
        b0_p = p[f'edge{s}_b0']
        W1_p = p[f'edge{s}_W1']
        if s == 0:
            tab, ce, Weff = _prep_first(nodes_l, g_l, Ws, Wr, Wg_p,
                                        b0_p, emb_edge_W,
                                        emb_edge_b, We_p)
        else:
            tab, ce = _prep(nodes_l, g_l, Ws, Wr, Wg_p, b0_p)
        G = [_gather2(tab, gidx[c]) for c in range(NCHUNK)]
        ea = [None, None]
        for c in range(NCHUNK):
            if s == 0:
                e_chunks[c], ea[c] = _edge_update(
                    edgesT, G[c], Weff, W1_p, ce, p['edge0_b1'],
                    transposed_e=True, col0=c * EC)
            else:
                e_chunks[c], ea[c] = _edge_update(
                    e_chunks[c], G[c], We_p, W1_p, ce,
                    p[f'edge{s}_b1'])
        ss = [_segsum2(e_chunks[c], sidx[c], ridx[c]) for c in range(NCHUNK)]
        W0n = p[f'node{s}_W0']
        Wn, Wse, Wre, Wgn = (W0n[i * LATENT:(i + 1) * LATENT] for i in range(4))
        cn = jnp.dot(g_l, Wgn) + p[f'node{s}_b0'].reshape(1, LATENT)
        W0g = p[f'glob{s}_W0']
        Wa, Wb, Wc = (W0g[i * LATENT:(i + 1) * LATENT] for i in range(3))
        nodes_l, na, g_l = _node_update(
            nodes_l, ss[0][0], ss[1][0], ss[0][1], ss[1][1],
            Wn, Wse, Wre, cn, p[f'node{s}_W1'], p[f'node{s}_b1'],
            ea[0], ea[1], g_l, Wa, Wb, Wc, p[f'glob{s}_b0'],
            p[f'glob{s}_W1'], p[f'glob{s}_b1'])
    edges_out = jnp.concatenate(e_chunks, axis=0)
    return nodes_l, edges_out, g_l
